# Initial kernel scaffold; baseline (speedup 1.0000x reference)
import jax
import jax.numpy as jnp
from jax.experimental import pallas as pl


def kernel(token_embeddings, attention_weights, params):
    raise NotImplementedError("write your pallas kernel here")


# speedup vs baseline: 2.8251x; 2.8251x over previous
import functools
import math

import jax
import jax.numpy as jnp
from jax import lax
from jax.experimental import pallas as pl
from jax.experimental.pallas import tpu as pltpu
from jax.experimental.pallas import tpu_sc as plsc

D = 1024
D2 = 512
NH = 8
HD = D // NH
BT = 256


def _gelu(x):
    return 0.5 * x * (1.0 + lax.erf(x * (1.0 / math.sqrt(2.0))))


def _dot(a, b):
    return jnp.dot(a, b, preferred_element_type=jnp.float32)


def _sc_gather_rows(table, idx):
    n_rows = idx.shape[0]
    info = plsc.get_sparse_core_info()
    num_workers = info.num_cores * info.num_subcores
    rows_per_w = n_rows // num_workers
    chunk = 16
    n_chunks = rows_per_w // chunk
    mesh = plsc.VectorSubcoreMesh(core_axis_name="c", subcore_axis_name="s")

    @functools.partial(
        pl.kernel,
        mesh=mesh,
        out_type=jax.ShapeDtypeStruct((n_rows, D), jnp.float32),
        scratch_types=[
            pltpu.VMEM((chunk,), jnp.int32),
            pltpu.VMEM((chunk, D), jnp.float32),
            pltpu.SemaphoreType.DMA,
        ],
    )
    def k(table_hbm, idx_hbm, out_hbm, idx_v, rows_v, sem):
        wid = lax.axis_index("s") * info.num_cores + lax.axis_index("c")
        base = wid * rows_per_w
        for c in range(n_chunks):
            off = base + c * chunk
            pltpu.sync_copy(idx_hbm.at[pl.ds(off, chunk)], idx_v)
            pltpu.async_copy(table_hbm.at[idx_v], rows_v, sem).wait()
            pltpu.sync_copy(rows_v, out_hbm.at[pl.ds(off, chunk)])

    return k(table, idx)


def _k_experts012(offs_ref, x_ref,
                  mw1, mb1, mw2, mb2,
                  sw1, sb1, sw2, sb2,
                  aw1, ab1, aw2, ab2,
                  bw1, bb1, bw2, bb2, bw3, bb3,
                  cw1a, cw1b, cb1, cw2, cb2,
                  o_ref):
    b = pl.program_id(0)
    j = pl.program_id(1)
    row0 = j * BT
    o1 = offs_ref[b, 0]
    o2 = offs_ref[b, 1]
    o3 = offs_ref[b, 2]
    rid = row0 + lax.broadcasted_iota(jnp.int32, (BT, 1), 0)
    x = x_ref[0]

    @pl.when(row0 < o1)
    def _():
        h = jnp.maximum(_dot(x, mw1[...]) + mb1[...], 0.0)
        e0 = _dot(h, mw2[...]) + mb2[...]
        o_ref[0] = jnp.where(rid < o1, e0, o_ref[0])

    @pl.when((row0 < o2) & (row0 + BT > o1))
    def _():
        h = _gelu(_dot(x, sw1[...]) + sb1[...])
        e1 = _dot(h, sw2[...]) + sb2[...]
        o_ref[0] = jnp.where((rid >= o1) & (rid < o2), e1, o_ref[0])

    @pl.when((row0 < o3) & (row0 + BT > o2))
    def _():
        a = _dot(_gelu(_dot(x, aw1[...]) + ab1[...]), aw2[...]) + ab2[...]
        t = _gelu(_dot(x, bw1[...]) + bb1[...])
        t = _gelu(_dot(t, bw2[...]) + bb2[...])
        bb = _dot(t, bw3[...]) + bb3[...]
        h = _gelu(_dot(a, cw1a[...]) + _dot(bb, cw1b[...]) + cb1[...])
        e2 = _dot(h, cw2[...]) + cb2[...]
        o_ref[0] = jnp.where((rid >= o2) & (rid < o3), e2, o_ref[0])


def _k_ffn_residual(offs_ref, x_ref, w1, b1, w2, b2, o_ref):
    b = pl.program_id(0)

    @pl.when(offs_ref[b, 3] > offs_ref[b, 2])
    def _():
        x = x_ref[0]
        h = _gelu(_dot(x, w1[...]) + b1[...])
        o_ref[0] = x + _dot(h, w2[...]) + b2[...]


def _k_kv(offs_ref, x_ref, wk, bk, wv, bv, k_ref, v_ref):
    b = pl.program_id(0)

    @pl.when(offs_ref[b, 3] > offs_ref[b, 2])
    def _():
        x = x_ref[0]
        k_ref[0] = _dot(x, wk[...]) + bk[...]
        v_ref[0] = _dot(x, wv[...]) + bv[...]


def _k_attn_fin(offs_ref, x_ref, k_ref, v_ref, wq, bq, wo, bo,
                fw1, fb1, fw2, fb2, o_ref):
    b = pl.program_id(0)
    j = pl.program_id(1)

    @pl.when(j * BT + BT > offs_ref[b, 2])
    def _():
        x = x_ref[0]
        q = _dot(x, wq[...]) + bq[...]
        scale = jnp.sqrt(jnp.asarray(HD, jnp.float32))
        outs = []
        for h in range(NH):
            qh = q[:, h * HD:(h + 1) * HD]
            kh = k_ref[0][:, h * HD:(h + 1) * HD]
            vh = v_ref[0][:, h * HD:(h + 1) * HD]
            s = lax.dot_general(qh, kh, (((1,), (1,)), ((), ())),
                                preferred_element_type=jnp.float32) / scale
            m = jnp.max(s, axis=-1, keepdims=True)
            p = jnp.exp(s - m)
            p = p / jnp.sum(p, axis=-1, keepdims=True)
            outs.append(_dot(p, vh))
        ao = jnp.concatenate(outs, axis=1)
        y = _dot(ao, wo[...]) + bo[...]
        h1 = _gelu(_dot(y, fw1[...]) + fb1[...])
        o_ref[0] = _dot(h1, fw2[...]) + fb2[...]


def _k_combine(offs_ref, a_ref, t_ref, w1m, w1t, b1, w2, b2, o_ref):
    b = pl.program_id(0)
    j = pl.program_id(1)
    o1 = offs_ref[b, 0]
    o2 = offs_ref[b, 1]
    o3 = offs_ref[b, 2]
    rid = j * BT + lax.broadcasted_iota(jnp.int32, (BT, 1), 0)
    p = jnp.where(rid >= o3, t_ref[0], a_ref[0])
    tier = ((rid >= o1).astype(jnp.int32) + (rid >= o2).astype(jnp.int32)
            + (rid >= o3).astype(jnp.int32))
    cols = lax.broadcasted_iota(jnp.int32, (BT, 8), 1)
    onehot = (tier == cols).astype(jnp.float32)
    h = _gelu(_dot(p, w1m[...]) + _dot(onehot, w1t[...]) + b1[...])
    logit = _dot(h, w2[...]) + b2[...]
    scl = jax.nn.sigmoid(logit[:, 0:1])
    o_ref[0] = p * scl


def _row2d(v):
    return v.reshape(1, -1)


def _wspec(shape):
    return pl.BlockSpec(shape, lambda b, j, offs: tuple(0 for _ in shape))


def _rowspec(B, S):
    return pl.BlockSpec((1, BT, D), lambda b, j, offs: (b, j, 0))


def kernel(token_embeddings, attention_weights, params):
    te, aw, p = token_embeddings, attention_weights, params
    B, S, _ = te.shape
    nblk = S // BT

    token_attention = aw.sum(axis=-1)
    max_att = jnp.max(token_attention, axis=-1, keepdims=True)
    na = token_attention / (max_att + 1e-8)
    t_min = jax.nn.sigmoid(p["th_minimal"])
    t_std = jax.nn.sigmoid(p["th_standard"])
    t_enh = jax.nn.sigmoid(p["th_enhanced"])
    tiers = jnp.where(na >= t_enh, 3,
                      jnp.where(na >= t_std, 2, jnp.where(na >= t_min, 1, 0)))
    sort_idx = jnp.argsort(tiers, axis=-1, stable=True)
    inv_idx = jnp.argsort(sort_idx, axis=-1)
    counts = jnp.sum(tiers[:, None, :] == jnp.arange(4)[None, :, None],
                     axis=-1).astype(jnp.int32)
    o1 = counts[:, 0]
    o2 = o1 + counts[:, 1]
    o3 = o2 + counts[:, 2]
    offs = jnp.stack([o1, o2, o3, jnp.full_like(o1, S)], axis=1)

    batch_base = (jnp.arange(B, dtype=jnp.int32) * S)[:, None]
    gather_idx = (sort_idx.astype(jnp.int32) + batch_base).reshape(-1)
    ungather_idx = (inv_idx.astype(jnp.int32) + batch_base).reshape(-1)

    xs = _sc_gather_rows(te.reshape(B * S, D), gather_idx).reshape(B, S, D)

    grid = (B, nblk)
    row = _rowspec(B, S)
    fullrow = pl.BlockSpec((1, S, D), lambda b, j, offs: (b, 0, 0))

    def call(body, in_specs, out_shape, out_specs, args):
        return pl.pallas_call(
            body,
            grid_spec=pltpu.PrefetchScalarGridSpec(
                num_scalar_prefetch=1, grid=grid,
                in_specs=in_specs, out_specs=out_specs),
            out_shape=out_shape,
            compiler_params=pltpu.CompilerParams(
                dimension_semantics=("parallel", "arbitrary")),
        )(offs, *args)

    f32 = jnp.float32
    shp = lambda: jax.ShapeDtypeStruct((B, S, D), f32)

    e_args = [
        xs,
        p["min_W1"], _row2d(p["min_b1"]), p["min_W2"], _row2d(p["min_b2"]),
        p["std_W1"], _row2d(p["std_b1"]), p["std_W2"], _row2d(p["std_b2"]),
        p["enh_a_W1"], _row2d(p["enh_a_b1"]), p["enh_a_W2"], _row2d(p["enh_a_b2"]),
        p["enh_b_W1"], _row2d(p["enh_b_b1"]), p["enh_b_W2"], _row2d(p["enh_b_b2"]),
        p["enh_b_W3"], _row2d(p["enh_b_b3"]),
        p["enh_c_W1"][:D], p["enh_c_W1"][D:], _row2d(p["enh_c_b1"]),
        p["enh_c_W2"], _row2d(p["enh_c_b2"]),
    ]
    e_specs = [row] + [_wspec(a.shape) for a in e_args[1:]]
    out012 = call(_k_experts012, e_specs, shp(), row, e_args)

    x3 = xs
    for i in range(3):
        w1, b1 = p[f"prem_{i}_W1"], _row2d(p[f"prem_{i}_b1"])
        w2, b2 = p[f"prem_{i}_W2"], _row2d(p[f"prem_{i}_b2"])
        specs = [row, _wspec(w1.shape), _wspec(b1.shape),
                 _wspec(w2.shape), _wspec(b2.shape)]
        x3 = call(_k_ffn_residual, specs, shp(), row, [x3, w1, b1, w2, b2])

    wq = p["attn_Wqkv"][:, :D]
    wk = p["attn_Wqkv"][:, D:2 * D]
    wv = p["attn_Wqkv"][:, 2 * D:]
    bq = _row2d(p["attn_bqkv"][:D])
    bk = _row2d(p["attn_bqkv"][D:2 * D])
    bv = _row2d(p["attn_bqkv"][2 * D:])

    kv_specs = [row, _wspec(wk.shape), _wspec(bk.shape),
                _wspec(wv.shape), _wspec(bv.shape)]
    karr, varr = call(_k_kv, kv_specs, (shp(), shp()), (row, row),
                      [x3, wk, bk, wv, bv])

    a_args = [x3, karr, varr, wq, bq, p["attn_Wo"], _row2d(p["attn_bo"]),
              p["fin_W1"], _row2d(p["fin_b1"]), p["fin_W2"], _row2d(p["fin_b2"])]
    a_specs = [row, fullrow, fullrow] + [_wspec(a.shape) for a in a_args[3:]]
    out3 = call(_k_attn_fin, a_specs, shp(), row, a_args)

    w1m = p["cc_W1"][:D]
    w1t = jnp.pad(p["cc_W1"][D:], ((0, 4), (0, 0)))
    w2 = jnp.pad(p["cc_W2"], ((0, 0), (0, 127)))
    b2 = jnp.pad(_row2d(p["cc_b2"]), ((0, 0), (0, 127)))
    c_args = [out012, out3, w1m, w1t, _row2d(p["cc_b1"]), w2, b2]
    c_specs = [row, row] + [_wspec(a.shape) for a in c_args[2:]]
    ys = call(_k_combine, c_specs, shp(), row, c_args)

    out = _sc_gather_rows(ys.reshape(B * S, D), ungather_idx)
    return out.reshape(B, S, D)
